# baseline (speedup 1.0000x reference)
import functools

import jax
import jax.numpy as jnp
from jax import lax
from jax.experimental import pallas as pl
from jax.experimental.pallas import tpu as pltpu
from jax.experimental.pallas import tpu_sc as plsc

N = 10000
E = 320000
D = 128
NC = 2
NS = 16
LANES = 16
CH = 128
E_PAD = 327680
ROWS_TOT = E_PAD // CH
NR = 8
N_PAD = 10240
PAD_DST = N
A_RPT = N_PAD // NS
ROWS_PT = ROWS_TOT // (NC * NS)
BLKS = ROWS_PT // NR
C0_ROWS_PT = 160
C1_ROWS_PT = 0
C0_ROWS = C0_ROWS_PT * NS

_mesh = plsc.VectorSubcoreMesh(
    core_axis_name="c", subcore_axis_name="s", num_cores=NC, num_subcores=NS)


@functools.partial(
    pl.kernel,
    out_type=jax.ShapeDtypeStruct((NC, N_PAD, LANES), jnp.float32),
    mesh=_mesh,
    scratch_types=[
        pltpu.VMEM_SHARED((N_PAD, LANES), jnp.float32),
        pltpu.VMEM((NR, CH), jnp.int32),
        pltpu.VMEM((CH, LANES), jnp.float32),
        pltpu.SemaphoreType.DMA,
    ],
)
def _deg_kernel(dst_hbm, out_hbm, hist, didx, ones, ssem):
    c = lax.axis_index("c")
    s = lax.axis_index("s")
    t = c * NS + s
    one16 = jnp.full((LANES,), 1.0, jnp.float32)
    zero16 = jnp.zeros((LANES,), jnp.float32)

    def zfill(i, _):
        ones[i, :] = zero16
        return 0

    lax.fori_loop(0, CH, zfill, 0)
    for k in range(A_RPT // CH):
        pltpu.sync_copy(ones, hist.at[pl.ds(s * A_RPT + k * CH, CH)])

    def ofill(i, _):
        ones[i, :] = one16
        return 0

    lax.fori_loop(0, CH, ofill, 0)
    plsc.subcore_barrier()

    def blk(j, _):
        row0 = t * ROWS_PT + j * NR
        pltpu.sync_copy(dst_hbm.at[pl.ds(row0, NR)], didx)
        cps = [pltpu.async_copy(ones, hist.at[didx.at[r]], ssem, add=True)
               for r in range(NR)]
        for cp in cps:
            cp.wait()
        return 0

    lax.fori_loop(0, BLKS, blk, 0)
    plsc.subcore_barrier()

    off = s * A_RPT
    pltpu.sync_copy(hist.at[pl.ds(off, A_RPT)],
                    out_hbm.at[c, pl.ds(off, A_RPT)])


@functools.partial(
    pl.kernel,
    out_type=jax.ShapeDtypeStruct((NC, N_PAD, D), jnp.float32),
    mesh=_mesh,
    scratch_types=[
        pltpu.VMEM_SHARED((N_PAD, D), jnp.float32),
        pltpu.VMEM((NR, CH), jnp.int32),
        pltpu.VMEM((NR, CH), jnp.int32),
        pltpu.VMEM((CH, D), jnp.float32),
        pltpu.VMEM((CH, D), jnp.float32),
        pltpu.SemaphoreType.DMA,
        pltpu.SemaphoreType.DMA,
        pltpu.SemaphoreType.DMA,
        pltpu.SemaphoreType.DMA,
    ],
)
def _agg_kernel(src_hbm, dst_hbm, hs_hbm, out_hbm, acc, sidx, didx, rows0,
                rows1, gsem0, gsem1, ssem0, ssem1):
    c = lax.axis_index("c")
    s = lax.axis_index("s")
    zero16 = jnp.zeros((LANES,), jnp.float32)
    rows = (rows0, rows1)
    gsems = (gsem0, gsem1)
    ssems = (ssem0, ssem1)

    def zfill(i, _):
        for k in range(D // LANES):
            rows0[i, k * LANES:(k + 1) * LANES] = zero16
        return 0

    lax.fori_loop(0, CH, zfill, 0)
    for k in range(A_RPT // CH):
        pltpu.sync_copy(rows0, acc.at[pl.ds(s * A_RPT + k * CH, CH)])
    plsc.subcore_barrier()

    tile_base = jnp.where(c == 0, s * C0_ROWS_PT, C0_ROWS + s * C1_ROWS_PT)
    nblk = jnp.where(c == 0, C0_ROWS_PT // NR, C1_ROWS_PT // NR)

    def blk(j, _):
        row0 = tile_base + j * NR
        pltpu.sync_copy(src_hbm.at[pl.ds(row0, NR)], sidx)
        pltpu.sync_copy(dst_hbm.at[pl.ds(row0, NR)], didx)
        g = pltpu.async_copy(hs_hbm.at[sidx.at[0]], rows0, gsem0)
        sprev = None
        for r in range(NR):
            g.wait()
            p = r % 2
            scur = pltpu.async_copy(rows[p], acc.at[didx.at[r]], ssems[p],
                                    add=True)
            if sprev is not None:
                sprev.wait()
            if r + 1 < NR:
                g = pltpu.async_copy(hs_hbm.at[sidx.at[r + 1]], rows[1 - p],
                                     gsems[1 - p])
            sprev = scur
        sprev.wait()
        return 0

    lax.fori_loop(0, nblk, blk, 0)
    plsc.subcore_barrier()

    o = s * A_RPT
    pltpu.sync_copy(acc.at[pl.ds(o, A_RPT)], out_hbm.at[c, pl.ds(o, A_RPT)])


_BN = 2000


def _dinv_of(degp_ref):
    deg = degp_ref[0, :, 0] + degp_ref[1, :, 0] + 1.0
    return lax.rsqrt(deg)


def _tc1_body(x_ref, w_ref, degp_ref, out_ref):
    dinv = _dinv_of(degp_ref)
    h = jnp.dot(x_ref[...], w_ref[...], preferred_element_type=jnp.float32)
    out_ref[...] = h * dinv[:, None]


def _tc2_body(acc_ref, hs_ref, degp_ref, b_ref, w_ref, out_ref):
    dinv = _dinv_of(degp_ref)
    tot = acc_ref[0] + acc_ref[1] + hs_ref[...]
    z = jnp.maximum(dinv[:, None] * tot + b_ref[...][None, :], 0.0)
    h = jnp.dot(z, w_ref[...], preferred_element_type=jnp.float32)
    out_ref[...] = h * dinv[:, None]


def _tc3_body(acc_ref, hs_ref, degp_ref, b_ref, out_ref):
    dinv = _dinv_of(degp_ref)
    tot = acc_ref[0] + acc_ref[1] + hs_ref[...]
    out_ref[...] = jnp.maximum(dinv[:, None] * tot + b_ref[...][None, :], 0.0)


_acc_spec = pl.BlockSpec((NC, _BN, D), lambda i: (0, i, 0))
_hs_spec = pl.BlockSpec((_BN, D), lambda i: (i, 0))
_degp_spec = pl.BlockSpec((NC, _BN, LANES), lambda i: (0, i, 0))
_vec_spec = pl.BlockSpec((D,), lambda i: (0,))
_w_spec = pl.BlockSpec((D, D), lambda i: (0, 0))


def _tc1(x, w0, degp):
    return pl.pallas_call(
        _tc1_body,
        grid=(N // _BN,),
        in_specs=[_hs_spec, _w_spec, _degp_spec],
        out_specs=_hs_spec,
        out_shape=jax.ShapeDtypeStruct((N, D), jnp.float32),
    )(x, w0, degp)


def _tc2(acc0, hs0, degp, b0, w1):
    return pl.pallas_call(
        _tc2_body,
        grid=(N // _BN,),
        in_specs=[_acc_spec, _hs_spec, _degp_spec, _vec_spec, _w_spec],
        out_specs=_hs_spec,
        out_shape=jax.ShapeDtypeStruct((N, D), jnp.float32),
    )(acc0, hs0, degp, b0, w1)


def _tc3(acc1, hs1, degp, b1):
    return pl.pallas_call(
        _tc3_body,
        grid=(N // _BN,),
        in_specs=[_acc_spec, _hs_spec, _degp_spec, _vec_spec],
        out_specs=_hs_spec,
        out_shape=jax.ShapeDtypeStruct((N, D), jnp.float32),
    )(acc1, hs1, degp, b1)


def kernel(x, edge_index, W0, b0, W1, b1):
    ei = edge_index.astype(jnp.int32)
    pad = E_PAD - E
    src2 = jnp.concatenate([ei[0], jnp.zeros((pad,), jnp.int32)]).reshape(
        ROWS_TOT, CH)
    dst2 = jnp.concatenate([ei[1], jnp.full((pad,), PAD_DST, jnp.int32)]).reshape(
        ROWS_TOT, CH)
    degp = _deg_kernel(dst2)
    hs0 = _tc1(x, W0, degp)
    acc0 = _agg_kernel(src2, dst2, hs0)
    hs1 = _tc2(acc0, hs0, degp, b0, W1)
    acc1 = _agg_kernel(src2, dst2, hs1)
    return _tc3(acc1, hs1, degp, b1)


# speedup vs baseline: 1.2653x; 1.2653x over previous
import functools

import jax
import jax.numpy as jnp
from jax import lax
from jax.experimental import pallas as pl
from jax.experimental.pallas import tpu as pltpu
from jax.experimental.pallas import tpu_sc as plsc

N = 10000
E = 320000
D = 128
NC = 2
NS = 16
LANES = 16
CH = 128
E_PAD = 327680
ROWS_TOT = E_PAD // CH
NR = 8
N_PAD = 10240
PAD_DST = N
A_RPT = N_PAD // NS
ROWS_PT = ROWS_TOT // (NC * NS)
BLKS = ROWS_PT // NR
C0_ROWS_PT = 80
C1_ROWS_PT = 80
C0_ROWS = C0_ROWS_PT * NS

_mesh = plsc.VectorSubcoreMesh(
    core_axis_name="c", subcore_axis_name="s", num_cores=NC, num_subcores=NS)


@functools.partial(
    pl.kernel,
    out_type=jax.ShapeDtypeStruct((NC, N_PAD, LANES), jnp.float32),
    mesh=_mesh,
    scratch_types=[
        pltpu.VMEM_SHARED((N_PAD, LANES), jnp.float32),
        pltpu.VMEM((NR, CH), jnp.int32),
        pltpu.VMEM((CH, LANES), jnp.float32),
        pltpu.SemaphoreType.DMA,
    ],
)
def _deg_kernel(dst_hbm, out_hbm, hist, didx, ones, ssem):
    c = lax.axis_index("c")
    s = lax.axis_index("s")
    t = c * NS + s
    one16 = jnp.full((LANES,), 1.0, jnp.float32)
    zero16 = jnp.zeros((LANES,), jnp.float32)

    def zfill(i, _):
        ones[i, :] = zero16
        return 0

    lax.fori_loop(0, CH, zfill, 0)
    for k in range(A_RPT // CH):
        pltpu.sync_copy(ones, hist.at[pl.ds(s * A_RPT + k * CH, CH)])

    def ofill(i, _):
        ones[i, :] = one16
        return 0

    lax.fori_loop(0, CH, ofill, 0)
    plsc.subcore_barrier()

    def blk(j, _):
        row0 = t * ROWS_PT + j * NR
        pltpu.sync_copy(dst_hbm.at[pl.ds(row0, NR)], didx)
        cps = [pltpu.async_copy(ones, hist.at[didx.at[r]], ssem, add=True)
               for r in range(NR)]
        for cp in cps:
            cp.wait()
        return 0

    lax.fori_loop(0, BLKS, blk, 0)
    plsc.subcore_barrier()

    off = s * A_RPT
    pltpu.sync_copy(hist.at[pl.ds(off, A_RPT)],
                    out_hbm.at[c, pl.ds(off, A_RPT)])


@functools.partial(
    pl.kernel,
    out_type=jax.ShapeDtypeStruct((NC, N_PAD, D), jnp.float32),
    mesh=_mesh,
    scratch_types=[
        pltpu.VMEM_SHARED((N_PAD, D), jnp.float32),
        pltpu.VMEM((NR, CH), jnp.int32),
        pltpu.VMEM((NR, CH), jnp.int32),
        pltpu.VMEM((CH, D), jnp.float32),
        pltpu.VMEM((CH, D), jnp.float32),
        pltpu.SemaphoreType.DMA,
        pltpu.SemaphoreType.DMA,
        pltpu.SemaphoreType.DMA,
        pltpu.SemaphoreType.DMA,
    ],
)
def _agg_kernel(src_hbm, dst_hbm, hs_hbm, out_hbm, acc, sidx, didx, rows0,
                rows1, gsem0, gsem1, ssem0, ssem1):
    c = lax.axis_index("c")
    s = lax.axis_index("s")
    zero16 = jnp.zeros((LANES,), jnp.float32)
    rows = (rows0, rows1)
    gsems = (gsem0, gsem1)
    ssems = (ssem0, ssem1)

    def zfill(i, _):
        for k in range(D // LANES):
            rows0[i, k * LANES:(k + 1) * LANES] = zero16
        return 0

    lax.fori_loop(0, CH, zfill, 0)
    for k in range(A_RPT // CH):
        pltpu.sync_copy(rows0, acc.at[pl.ds(s * A_RPT + k * CH, CH)])
    plsc.subcore_barrier()

    tile_base = jnp.where(c == 0, s * C0_ROWS_PT, C0_ROWS + s * C1_ROWS_PT)
    nblk = jnp.where(c == 0, C0_ROWS_PT // NR, C1_ROWS_PT // NR)

    def blk(j, _):
        row0 = tile_base + j * NR
        pltpu.sync_copy(src_hbm.at[pl.ds(row0, NR)], sidx)
        pltpu.sync_copy(dst_hbm.at[pl.ds(row0, NR)], didx)
        g = pltpu.async_copy(hs_hbm.at[sidx.at[0]], rows0, gsem0)
        sprev = None
        for r in range(NR):
            g.wait()
            p = r % 2
            scur = pltpu.async_copy(rows[p], acc.at[didx.at[r]], ssems[p],
                                    add=True)
            if sprev is not None:
                sprev.wait()
            if r + 1 < NR:
                g = pltpu.async_copy(hs_hbm.at[sidx.at[r + 1]], rows[1 - p],
                                     gsems[1 - p])
            sprev = scur
        sprev.wait()
        return 0

    lax.fori_loop(0, nblk, blk, 0)
    plsc.subcore_barrier()

    o = s * A_RPT
    pltpu.sync_copy(acc.at[pl.ds(o, A_RPT)], out_hbm.at[c, pl.ds(o, A_RPT)])


_BN = 2000


def _dinv_of(degp_ref):
    deg = degp_ref[0, :, 0] + degp_ref[1, :, 0] + 1.0
    return lax.rsqrt(deg)


def _tc1_body(x_ref, w_ref, degp_ref, out_ref):
    dinv = _dinv_of(degp_ref)
    h = jnp.dot(x_ref[...], w_ref[...], preferred_element_type=jnp.float32)
    out_ref[...] = h * dinv[:, None]


def _tc2_body(acc_ref, hs_ref, degp_ref, b_ref, w_ref, out_ref):
    dinv = _dinv_of(degp_ref)
    tot = acc_ref[0] + acc_ref[1] + hs_ref[...]
    z = jnp.maximum(dinv[:, None] * tot + b_ref[...][None, :], 0.0)
    h = jnp.dot(z, w_ref[...], preferred_element_type=jnp.float32)
    out_ref[...] = h * dinv[:, None]


def _tc3_body(acc_ref, hs_ref, degp_ref, b_ref, out_ref):
    dinv = _dinv_of(degp_ref)
    tot = acc_ref[0] + acc_ref[1] + hs_ref[...]
    out_ref[...] = jnp.maximum(dinv[:, None] * tot + b_ref[...][None, :], 0.0)


_acc_spec = pl.BlockSpec((NC, _BN, D), lambda i: (0, i, 0))
_hs_spec = pl.BlockSpec((_BN, D), lambda i: (i, 0))
_degp_spec = pl.BlockSpec((NC, _BN, LANES), lambda i: (0, i, 0))
_vec_spec = pl.BlockSpec((D,), lambda i: (0,))
_w_spec = pl.BlockSpec((D, D), lambda i: (0, 0))


def _tc1(x, w0, degp):
    return pl.pallas_call(
        _tc1_body,
        grid=(N // _BN,),
        in_specs=[_hs_spec, _w_spec, _degp_spec],
        out_specs=_hs_spec,
        out_shape=jax.ShapeDtypeStruct((N, D), jnp.float32),
    )(x, w0, degp)


def _tc2(acc0, hs0, degp, b0, w1):
    return pl.pallas_call(
        _tc2_body,
        grid=(N // _BN,),
        in_specs=[_acc_spec, _hs_spec, _degp_spec, _vec_spec, _w_spec],
        out_specs=_hs_spec,
        out_shape=jax.ShapeDtypeStruct((N, D), jnp.float32),
    )(acc0, hs0, degp, b0, w1)


def _tc3(acc1, hs1, degp, b1):
    return pl.pallas_call(
        _tc3_body,
        grid=(N // _BN,),
        in_specs=[_acc_spec, _hs_spec, _degp_spec, _vec_spec],
        out_specs=_hs_spec,
        out_shape=jax.ShapeDtypeStruct((N, D), jnp.float32),
    )(acc1, hs1, degp, b1)


def kernel(x, edge_index, W0, b0, W1, b1):
    ei = edge_index.astype(jnp.int32)
    pad = E_PAD - E
    src2 = jnp.concatenate([ei[0], jnp.zeros((pad,), jnp.int32)]).reshape(
        ROWS_TOT, CH)
    padv = N + jnp.arange(pad, dtype=jnp.int32) % (N_PAD - N)
    dst2 = jnp.concatenate([ei[1], padv]).reshape(ROWS_TOT, CH)
    degp = _deg_kernel(dst2)
    hs0 = _tc1(x, W0, degp)
    acc0 = _agg_kernel(src2, dst2, hs0)
    hs1 = _tc2(acc0, hs0, degp, b0, W1)
    acc1 = _agg_kernel(src2, dst2, hs1)
    return _tc3(acc1, hs1, degp, b1)


# speedup vs baseline: 3.7021x; 2.9259x over previous
import functools

import jax
import jax.numpy as jnp
from jax import lax
from jax.experimental import pallas as pl
from jax.experimental.pallas import tpu as pltpu
from jax.experimental.pallas import tpu_sc as plsc

N = 10000
E = 320000
D = 128
NC = 2
NS = 16
LANES = 16
CH = 128
E_PAD = 327680
ROWS_TOT = E_PAD // CH
NR = 8
N_PAD = 10240
PAD_DST = N
A_RPT = N_PAD // NS
ROWS_PT = ROWS_TOT // (NC * NS)
BLKS = ROWS_PT // NR
C0_ROWS_PT = 80
C1_ROWS_PT = 80
C0_ROWS = C0_ROWS_PT * NS

_mesh = plsc.VectorSubcoreMesh(
    core_axis_name="c", subcore_axis_name="s", num_cores=NC, num_subcores=NS)


@functools.partial(
    pl.kernel,
    out_type=jax.ShapeDtypeStruct((NC, N_PAD, LANES), jnp.float32),
    mesh=_mesh,
    scratch_types=[
        pltpu.VMEM_SHARED((N_PAD, LANES), jnp.float32),
        pltpu.VMEM((NR, CH), jnp.int32),
        pltpu.VMEM((CH, LANES), jnp.float32),
        pltpu.SemaphoreType.DMA,
    ],
)
def _deg_kernel(dst_hbm, out_hbm, hist, didx, ones, ssem):
    c = lax.axis_index("c")
    s = lax.axis_index("s")
    t = c * NS + s
    one16 = jnp.full((LANES,), 1.0, jnp.float32)
    zero16 = jnp.zeros((LANES,), jnp.float32)

    def zfill(i, _):
        ones[i, :] = zero16
        return 0

    lax.fori_loop(0, CH, zfill, 0)
    for k in range(A_RPT // CH):
        pltpu.sync_copy(ones, hist.at[pl.ds(s * A_RPT + k * CH, CH)])

    def ofill(i, _):
        ones[i, :] = one16
        return 0

    lax.fori_loop(0, CH, ofill, 0)
    plsc.subcore_barrier()

    def blk(j, _):
        row0 = t * ROWS_PT + j * NR
        pltpu.sync_copy(dst_hbm.at[pl.ds(row0, NR)], didx)
        cps = [pltpu.async_copy(ones, hist.at[didx.at[r]], ssem, add=True)
               for r in range(NR)]
        for cp in cps:
            cp.wait()
        return 0

    lax.fori_loop(0, BLKS, blk, 0)
    plsc.subcore_barrier()

    off = s * A_RPT
    pltpu.sync_copy(hist.at[pl.ds(off, A_RPT)],
                    out_hbm.at[c, pl.ds(off, A_RPT)])


@functools.partial(
    pl.kernel,
    out_type=jax.ShapeDtypeStruct((NC, N_PAD, D), jnp.float32),
    mesh=_mesh,
    scratch_types=[
        pltpu.VMEM_SHARED((N_PAD, D), jnp.float32),
        pltpu.VMEM((NR, CH), jnp.int32),
        pltpu.VMEM((NR, CH), jnp.int32),
        pltpu.VMEM((CH, D), jnp.float32),
        pltpu.VMEM((CH, D), jnp.float32),
        pltpu.SemaphoreType.DMA,
        pltpu.SemaphoreType.DMA,
        pltpu.SemaphoreType.DMA,
        pltpu.SemaphoreType.DMA,
    ],
)
def _agg_kernel(src_hbm, dst_hbm, hs_hbm, out_hbm, acc, sidx, didx, rows0,
                rows1, gsem0, gsem1, ssem0, ssem1):
    c = lax.axis_index("c")
    s = lax.axis_index("s")
    zero16 = jnp.zeros((LANES,), jnp.float32)
    rows = (rows0, rows1)
    gsems = (gsem0, gsem1)
    ssems = (ssem0, ssem1)

    def zfill(i, _):
        for k in range(D // LANES):
            rows0[i, k * LANES:(k + 1) * LANES] = zero16
        return 0

    lax.fori_loop(0, CH, zfill, 0)
    for k in range(A_RPT // CH):
        pltpu.sync_copy(rows0, acc.at[pl.ds(s * A_RPT + k * CH, CH)])
    plsc.subcore_barrier()

    tile_base = jnp.where(c == 0, s * C0_ROWS_PT, C0_ROWS + s * C1_ROWS_PT)
    nblk = jnp.where(c == 0, C0_ROWS_PT // NR, C1_ROWS_PT // NR)

    def blk(j, _):
        row0 = tile_base + j * NR
        pltpu.sync_copy(src_hbm.at[pl.ds(row0, NR)], sidx)
        pltpu.sync_copy(dst_hbm.at[pl.ds(row0, NR)], didx)
        g = pltpu.async_copy(hs_hbm.at[sidx.at[0]], rows0, gsem0)
        sprev = None
        for r in range(NR):
            g.wait()
            p = r % 2
            scur = pltpu.async_copy(rows[p], acc.at[didx.at[r]], ssems[p],
                                    add=True)
            if sprev is not None:
                sprev.wait()
            if r + 1 < NR:
                g = pltpu.async_copy(hs_hbm.at[sidx.at[r + 1]], rows[1 - p],
                                     gsems[1 - p])
            sprev = scur
        sprev.wait()
        return 0

    lax.fori_loop(0, nblk, blk, 0)
    plsc.subcore_barrier()

    o = s * A_RPT
    pltpu.sync_copy(acc.at[pl.ds(o, A_RPT)], out_hbm.at[c, pl.ds(o, A_RPT)])


_BN = 2000


def _dinv_of(degp_ref):
    deg = degp_ref[0, :, 0] + degp_ref[1, :, 0] + 1.0
    return lax.rsqrt(deg)


def _tc1_body(x_ref, w_ref, degp_ref, out_ref):
    dinv = _dinv_of(degp_ref)
    h = jnp.dot(x_ref[...], w_ref[...], preferred_element_type=jnp.float32)
    out_ref[...] = h * dinv[:, None]


def _tc2_body(acc_ref, hs_ref, degp_ref, b_ref, w_ref, out_ref):
    dinv = _dinv_of(degp_ref)
    tot = acc_ref[0] + acc_ref[1] + hs_ref[...]
    z = jnp.maximum(dinv[:, None] * tot + b_ref[...][None, :], 0.0)
    h = jnp.dot(z, w_ref[...], preferred_element_type=jnp.float32)
    out_ref[...] = h * dinv[:, None]


def _tc3_body(acc_ref, hs_ref, degp_ref, b_ref, out_ref):
    dinv = _dinv_of(degp_ref)
    tot = acc_ref[0] + acc_ref[1] + hs_ref[...]
    out_ref[...] = jnp.maximum(dinv[:, None] * tot + b_ref[...][None, :], 0.0)


_acc_spec = pl.BlockSpec((NC, _BN, D), lambda i: (0, i, 0))
_hs_spec = pl.BlockSpec((_BN, D), lambda i: (i, 0))
_degp_spec = pl.BlockSpec((NC, _BN, LANES), lambda i: (0, i, 0))
_vec_spec = pl.BlockSpec((D,), lambda i: (0,))
_w_spec = pl.BlockSpec((D, D), lambda i: (0, 0))


def _tc1(x, w0, degp):
    return pl.pallas_call(
        _tc1_body,
        grid=(N // _BN,),
        in_specs=[_hs_spec, _w_spec, _degp_spec],
        out_specs=_hs_spec,
        out_shape=jax.ShapeDtypeStruct((N, D), jnp.float32),
    )(x, w0, degp)


def _tc2(acc0, hs0, degp, b0, w1):
    return pl.pallas_call(
        _tc2_body,
        grid=(N // _BN,),
        in_specs=[_acc_spec, _hs_spec, _degp_spec, _vec_spec, _w_spec],
        out_specs=_hs_spec,
        out_shape=jax.ShapeDtypeStruct((N, D), jnp.float32),
    )(acc0, hs0, degp, b0, w1)


def _tc3(acc1, hs1, degp, b1):
    return pl.pallas_call(
        _tc3_body,
        grid=(N // _BN,),
        in_specs=[_acc_spec, _hs_spec, _degp_spec, _vec_spec],
        out_specs=_hs_spec,
        out_shape=jax.ShapeDtypeStruct((N, D), jnp.float32),
    )(acc1, hs1, degp, b1)


def kernel(x, edge_index, W0, b0, W1, b1):
    ei = edge_index.astype(jnp.int32)
    pad = E_PAD - E
    src2 = jnp.concatenate([ei[0], jnp.arange(pad, dtype=jnp.int32)]).reshape(
        ROWS_TOT, CH)
    padv = N + jnp.arange(pad, dtype=jnp.int32) % (N_PAD - N)
    dst2 = jnp.concatenate([ei[1], padv]).reshape(ROWS_TOT, CH)
    degp = _deg_kernel(dst2)
    hs0 = _tc1(x, W0, degp)
    acc0 = _agg_kernel(src2, dst2, hs0)
    hs1 = _tc2(acc0, hs0, degp, b0, W1)
    acc1 = _agg_kernel(src2, dst2, hs1)
    return _tc3(acc1, hs1, degp, b1)


# speedup vs baseline: 3.7866x; 1.0228x over previous
import functools

import jax
import jax.numpy as jnp
from jax import lax
from jax.experimental import pallas as pl
from jax.experimental.pallas import tpu as pltpu
from jax.experimental.pallas import tpu_sc as plsc

N = 10000
E = 320000
D = 128
NC = 2
NS = 16
LANES = 16
CH = 64
E_PAD = 327680
ROWS_TOT = E_PAD // CH
NR = 16
N_PAD = 10240
PAD_DST = N
A_RPT = N_PAD // NS
ROWS_PT = ROWS_TOT // (NC * NS)
BLKS = ROWS_PT // NR
C0_ROWS_PT = 160
C1_ROWS_PT = 160
C0_ROWS = C0_ROWS_PT * NS

_mesh = plsc.VectorSubcoreMesh(
    core_axis_name="c", subcore_axis_name="s", num_cores=NC, num_subcores=NS)


@functools.partial(
    pl.kernel,
    out_type=jax.ShapeDtypeStruct((NC, N_PAD, LANES), jnp.float32),
    mesh=_mesh,
    scratch_types=[
        pltpu.VMEM_SHARED((N_PAD, LANES), jnp.float32),
        pltpu.VMEM((NR, CH), jnp.int32),
        pltpu.VMEM((CH, LANES), jnp.float32),
        pltpu.SemaphoreType.DMA,
    ],
)
def _deg_kernel(dst_hbm, out_hbm, hist, didx, ones, ssem):
    c = lax.axis_index("c")
    s = lax.axis_index("s")
    t = c * NS + s
    one16 = jnp.full((LANES,), 1.0, jnp.float32)
    zero16 = jnp.zeros((LANES,), jnp.float32)

    def zfill(i, _):
        ones[i, :] = zero16
        return 0

    lax.fori_loop(0, CH, zfill, 0)
    for k in range(A_RPT // CH):
        pltpu.sync_copy(ones, hist.at[pl.ds(s * A_RPT + k * CH, CH)])

    def ofill(i, _):
        ones[i, :] = one16
        return 0

    lax.fori_loop(0, CH, ofill, 0)
    plsc.subcore_barrier()

    def blk(j, _):
        row0 = t * ROWS_PT + j * NR
        pltpu.sync_copy(dst_hbm.at[pl.ds(row0, NR)], didx)
        cps = [pltpu.async_copy(ones, hist.at[didx.at[r]], ssem, add=True)
               for r in range(NR)]
        for cp in cps:
            cp.wait()
        return 0

    lax.fori_loop(0, BLKS, blk, 0)
    plsc.subcore_barrier()

    off = s * A_RPT
    pltpu.sync_copy(hist.at[pl.ds(off, A_RPT)],
                    out_hbm.at[c, pl.ds(off, A_RPT)])


@functools.partial(
    pl.kernel,
    out_type=jax.ShapeDtypeStruct((NC, N_PAD, D), jnp.float32),
    mesh=_mesh,
    scratch_types=[
        pltpu.VMEM_SHARED((N_PAD, D), jnp.float32),
        pltpu.VMEM((NR, CH), jnp.int32),
        pltpu.VMEM((NR, CH), jnp.int32),
        pltpu.VMEM((CH, D), jnp.float32),
        pltpu.VMEM((CH, D), jnp.float32),
        pltpu.VMEM((CH, D), jnp.float32),
        pltpu.VMEM((CH, D), jnp.float32),
        pltpu.SemaphoreType.DMA,
        pltpu.SemaphoreType.DMA,
        pltpu.SemaphoreType.DMA,
        pltpu.SemaphoreType.DMA,
        pltpu.SemaphoreType.DMA,
        pltpu.SemaphoreType.DMA,
        pltpu.SemaphoreType.DMA,
        pltpu.SemaphoreType.DMA,
    ],
)
def _agg_kernel(src_hbm, dst_hbm, hs_hbm, out_hbm, acc, sidx, didx, rows0,
                rows1, rows2, rows3, gsem0, gsem1, gsem2, gsem3, ssem0, ssem1,
                ssem2, ssem3):
    c = lax.axis_index("c")
    s = lax.axis_index("s")
    zero16 = jnp.zeros((LANES,), jnp.float32)
    rows = (rows0, rows1, rows2, rows3)
    gsems = (gsem0, gsem1, gsem2, gsem3)
    ssems = (ssem0, ssem1, ssem2, ssem3)
    NB = 4

    def zfill(i, _):
        for k in range(D // LANES):
            rows0[i, k * LANES:(k + 1) * LANES] = zero16
        return 0

    lax.fori_loop(0, CH, zfill, 0)
    for k in range(A_RPT // CH):
        pltpu.sync_copy(rows0, acc.at[pl.ds(s * A_RPT + k * CH, CH)])
    plsc.subcore_barrier()

    tile_base = jnp.where(c == 0, s * C0_ROWS_PT, C0_ROWS + s * C1_ROWS_PT)
    nblk = jnp.where(c == 0, C0_ROWS_PT // NR, C1_ROWS_PT // NR)

    def blk(j, _):
        row0 = tile_base + j * NR
        pltpu.sync_copy(src_hbm.at[pl.ds(row0, NR)], sidx)
        pltpu.sync_copy(dst_hbm.at[pl.ds(row0, NR)], didx)
        gd = {}
        sd = {}
        gd[0] = pltpu.async_copy(hs_hbm.at[sidx.at[0]], rows[0], gsems[0])
        for r in range(NR):
            if r + 1 < NR:
                nb = (r + 1) % NB
                if r + 1 >= NB:
                    sd[r + 1 - NB].wait()
                gd[r + 1] = pltpu.async_copy(hs_hbm.at[sidx.at[r + 1]],
                                             rows[nb], gsems[nb])
            gd[r].wait()
            b = r % NB
            sd[r] = pltpu.async_copy(rows[b], acc.at[didx.at[r]], ssems[b],
                                     add=True)
        for r in range(max(0, NR - NB), NR):
            sd[r].wait()
        return 0

    lax.fori_loop(0, nblk, blk, 0)
    plsc.subcore_barrier()

    o = s * A_RPT
    pltpu.sync_copy(acc.at[pl.ds(o, A_RPT)], out_hbm.at[c, pl.ds(o, A_RPT)])


_BN = 2000


def _dinv_of(degp_ref):
    deg = degp_ref[0, :, 0] + degp_ref[1, :, 0] + 1.0
    return lax.rsqrt(deg)


def _tc1_body(x_ref, w_ref, degp_ref, out_ref):
    dinv = _dinv_of(degp_ref)
    h = jnp.dot(x_ref[...], w_ref[...], preferred_element_type=jnp.float32)
    out_ref[...] = h * dinv[:, None]


def _tc2_body(acc_ref, hs_ref, degp_ref, b_ref, w_ref, out_ref):
    dinv = _dinv_of(degp_ref)
    tot = acc_ref[0] + acc_ref[1] + hs_ref[...]
    z = jnp.maximum(dinv[:, None] * tot + b_ref[...][None, :], 0.0)
    h = jnp.dot(z, w_ref[...], preferred_element_type=jnp.float32)
    out_ref[...] = h * dinv[:, None]


def _tc3_body(acc_ref, hs_ref, degp_ref, b_ref, out_ref):
    dinv = _dinv_of(degp_ref)
    tot = acc_ref[0] + acc_ref[1] + hs_ref[...]
    out_ref[...] = jnp.maximum(dinv[:, None] * tot + b_ref[...][None, :], 0.0)


_acc_spec = pl.BlockSpec((NC, _BN, D), lambda i: (0, i, 0))
_hs_spec = pl.BlockSpec((_BN, D), lambda i: (i, 0))
_degp_spec = pl.BlockSpec((NC, _BN, LANES), lambda i: (0, i, 0))
_vec_spec = pl.BlockSpec((D,), lambda i: (0,))
_w_spec = pl.BlockSpec((D, D), lambda i: (0, 0))


def _tc1(x, w0, degp):
    return pl.pallas_call(
        _tc1_body,
        grid=(N // _BN,),
        in_specs=[_hs_spec, _w_spec, _degp_spec],
        out_specs=_hs_spec,
        out_shape=jax.ShapeDtypeStruct((N, D), jnp.float32),
    )(x, w0, degp)


def _tc2(acc0, hs0, degp, b0, w1):
    return pl.pallas_call(
        _tc2_body,
        grid=(N // _BN,),
        in_specs=[_acc_spec, _hs_spec, _degp_spec, _vec_spec, _w_spec],
        out_specs=_hs_spec,
        out_shape=jax.ShapeDtypeStruct((N, D), jnp.float32),
    )(acc0, hs0, degp, b0, w1)


def _tc3(acc1, hs1, degp, b1):
    return pl.pallas_call(
        _tc3_body,
        grid=(N // _BN,),
        in_specs=[_acc_spec, _hs_spec, _degp_spec, _vec_spec],
        out_specs=_hs_spec,
        out_shape=jax.ShapeDtypeStruct((N, D), jnp.float32),
    )(acc1, hs1, degp, b1)


def kernel(x, edge_index, W0, b0, W1, b1):
    ei = edge_index.astype(jnp.int32)
    pad = E_PAD - E
    src2 = jnp.concatenate([ei[0], jnp.arange(pad, dtype=jnp.int32)]).reshape(
        ROWS_TOT, CH)
    padv = N + jnp.arange(pad, dtype=jnp.int32) % (N_PAD - N)
    dst2 = jnp.concatenate([ei[1], padv]).reshape(ROWS_TOT, CH)
    degp = _deg_kernel(dst2)
    hs0 = _tc1(x, W0, degp)
    acc0 = _agg_kernel(src2, dst2, hs0)
    hs1 = _tc2(acc0, hs0, degp, b0, W1)
    acc1 = _agg_kernel(src2, dst2, hs1)
    return _tc3(acc1, hs1, degp, b1)


# speedup vs baseline: 4.0043x; 1.0575x over previous
import functools

import jax
import jax.numpy as jnp
from jax import lax
from jax.experimental import pallas as pl
from jax.experimental.pallas import tpu as pltpu
from jax.experimental.pallas import tpu_sc as plsc

N = 10000
E = 320000
D = 128
NC = 2
NS = 16
LANES = 16
CH = 128
E_PAD = 327680
ROWS_TOT = E_PAD // CH
NR = 8
N_PAD = 10240
PAD_DST = N
A_RPT = N_PAD // NS
ROWS_PT = ROWS_TOT // (NC * NS)
BLKS = ROWS_PT // NR
C0_ROWS_PT = 80
C1_ROWS_PT = 80
C0_ROWS = C0_ROWS_PT * NS

_mesh = plsc.VectorSubcoreMesh(
    core_axis_name="c", subcore_axis_name="s", num_cores=NC, num_subcores=NS)


@functools.partial(
    pl.kernel,
    out_type=jax.ShapeDtypeStruct((NC, N_PAD, LANES), jnp.float32),
    mesh=_mesh,
    scratch_types=[
        pltpu.VMEM_SHARED((N_PAD, LANES), jnp.float32),
        pltpu.VMEM((NR, CH), jnp.int32),
        pltpu.VMEM((CH, LANES), jnp.float32),
        pltpu.SemaphoreType.DMA,
    ],
)
def _deg_kernel(dst_hbm, out_hbm, hist, didx, ones, ssem):
    c = lax.axis_index("c")
    s = lax.axis_index("s")
    t = c * NS + s
    one16 = jnp.full((LANES,), 1.0, jnp.float32)
    zero16 = jnp.zeros((LANES,), jnp.float32)

    def zfill(i, _):
        ones[i, :] = zero16
        return 0

    lax.fori_loop(0, CH, zfill, 0)
    for k in range(A_RPT // CH):
        pltpu.sync_copy(ones, hist.at[pl.ds(s * A_RPT + k * CH, CH)])

    def ofill(i, _):
        ones[i, :] = one16
        return 0

    lax.fori_loop(0, CH, ofill, 0)
    plsc.subcore_barrier()

    def blk(j, _):
        row0 = t * ROWS_PT + j * NR
        pltpu.sync_copy(dst_hbm.at[pl.ds(row0, NR)], didx)
        cps = [pltpu.async_copy(ones, hist.at[didx.at[r]], ssem, add=True)
               for r in range(NR)]
        for cp in cps:
            cp.wait()
        return 0

    lax.fori_loop(0, BLKS, blk, 0)
    plsc.subcore_barrier()

    off = s * A_RPT
    pltpu.sync_copy(hist.at[pl.ds(off, A_RPT)],
                    out_hbm.at[c, pl.ds(off, A_RPT)])


@functools.partial(
    pl.kernel,
    out_type=jax.ShapeDtypeStruct((NC, N_PAD, D), jnp.float32),
    mesh=_mesh,
    scratch_types=[
        pltpu.VMEM_SHARED((N_PAD, D), jnp.float32),
        pltpu.VMEM((NR, CH), jnp.int32),
        pltpu.VMEM((NR, CH), jnp.int32),
        pltpu.VMEM((CH, D), jnp.float32),
        pltpu.VMEM((CH, D), jnp.float32),
        pltpu.SemaphoreType.DMA,
        pltpu.SemaphoreType.DMA,
        pltpu.SemaphoreType.DMA,
        pltpu.SemaphoreType.DMA,
    ],
)
def _agg_kernel(src_hbm, dst_hbm, hs_hbm, out_hbm, acc, sidx, didx, rows0,
                rows1, gsem0, gsem1, ssem0, ssem1):
    c = lax.axis_index("c")
    s = lax.axis_index("s")
    zero16 = jnp.zeros((LANES,), jnp.float32)
    rows = (rows0, rows1)
    gsems = (gsem0, gsem1)
    ssems = (ssem0, ssem1)
    NB = 2

    def zfill(i, _):
        for k in range(D // LANES):
            rows0[i, k * LANES:(k + 1) * LANES] = zero16
        return 0

    lax.fori_loop(0, CH, zfill, 0)
    for k in range(A_RPT // CH):
        pltpu.sync_copy(rows0, acc.at[pl.ds(s * A_RPT + k * CH, CH)])
    plsc.subcore_barrier()

    tile_base = jnp.where(c == 0, s * C0_ROWS_PT, C0_ROWS + s * C1_ROWS_PT)
    nblk = jnp.where(c == 0, C0_ROWS_PT // NR, C1_ROWS_PT // NR)

    def blk(j, _):
        row0 = tile_base + j * NR
        pltpu.sync_copy(src_hbm.at[pl.ds(row0, NR)], sidx)
        pltpu.sync_copy(dst_hbm.at[pl.ds(row0, NR)], didx)
        gd = {}
        sd = {}
        gd[0] = pltpu.async_copy(hs_hbm.at[sidx.at[0]], rows[0], gsems[0])
        for r in range(NR):
            if r + 1 < NR:
                nb = (r + 1) % NB
                if r >= 1:
                    sd[r - 1].wait()
                gd[r + 1] = pltpu.async_copy(hs_hbm.at[sidx.at[r + 1]],
                                             rows[nb], gsems[nb])
            gd[r].wait()
            b = r % NB
            sd[r] = pltpu.async_copy(rows[b], acc.at[didx.at[r]], ssems[b],
                                     add=True)
        for r in range(max(0, NR - NB), NR):
            sd[r].wait()
        return 0

    lax.fori_loop(0, nblk, blk, 0)
    plsc.subcore_barrier()

    o = s * A_RPT
    pltpu.sync_copy(acc.at[pl.ds(o, A_RPT)], out_hbm.at[c, pl.ds(o, A_RPT)])


_BN = 2000


def _dinv_of(degp_ref):
    deg = degp_ref[0, :, 0] + degp_ref[1, :, 0] + 1.0
    return lax.rsqrt(deg)


def _tc1_body(x_ref, w_ref, degp_ref, out_ref):
    dinv = _dinv_of(degp_ref)
    h = jnp.dot(x_ref[...], w_ref[...], preferred_element_type=jnp.float32)
    out_ref[...] = h * dinv[:, None]


def _tc2_body(acc_ref, hs_ref, degp_ref, b_ref, w_ref, out_ref):
    dinv = _dinv_of(degp_ref)
    tot = acc_ref[0] + acc_ref[1] + hs_ref[...]
    z = jnp.maximum(dinv[:, None] * tot + b_ref[...][None, :], 0.0)
    h = jnp.dot(z, w_ref[...], preferred_element_type=jnp.float32)
    out_ref[...] = h * dinv[:, None]


def _tc3_body(acc_ref, hs_ref, degp_ref, b_ref, out_ref):
    dinv = _dinv_of(degp_ref)
    tot = acc_ref[0] + acc_ref[1] + hs_ref[...]
    out_ref[...] = jnp.maximum(dinv[:, None] * tot + b_ref[...][None, :], 0.0)


_acc_spec = pl.BlockSpec((NC, _BN, D), lambda i: (0, i, 0))
_hs_spec = pl.BlockSpec((_BN, D), lambda i: (i, 0))
_degp_spec = pl.BlockSpec((NC, _BN, LANES), lambda i: (0, i, 0))
_vec_spec = pl.BlockSpec((D,), lambda i: (0,))
_w_spec = pl.BlockSpec((D, D), lambda i: (0, 0))


def _tc1(x, w0, degp):
    return pl.pallas_call(
        _tc1_body,
        grid=(N // _BN,),
        in_specs=[_hs_spec, _w_spec, _degp_spec],
        out_specs=_hs_spec,
        out_shape=jax.ShapeDtypeStruct((N, D), jnp.float32),
    )(x, w0, degp)


def _tc2(acc0, hs0, degp, b0, w1):
    return pl.pallas_call(
        _tc2_body,
        grid=(N // _BN,),
        in_specs=[_acc_spec, _hs_spec, _degp_spec, _vec_spec, _w_spec],
        out_specs=_hs_spec,
        out_shape=jax.ShapeDtypeStruct((N, D), jnp.float32),
    )(acc0, hs0, degp, b0, w1)


def _tc3(acc1, hs1, degp, b1):
    return pl.pallas_call(
        _tc3_body,
        grid=(N // _BN,),
        in_specs=[_acc_spec, _hs_spec, _degp_spec, _vec_spec],
        out_specs=_hs_spec,
        out_shape=jax.ShapeDtypeStruct((N, D), jnp.float32),
    )(acc1, hs1, degp, b1)


def kernel(x, edge_index, W0, b0, W1, b1):
    ei = edge_index.astype(jnp.int32)
    pad = E_PAD - E
    src2 = jnp.concatenate([ei[0], jnp.arange(pad, dtype=jnp.int32)]).reshape(
        ROWS_TOT, CH)
    padv = N + jnp.arange(pad, dtype=jnp.int32) % (N_PAD - N)
    dst2 = jnp.concatenate([ei[1], padv]).reshape(ROWS_TOT, CH)
    degp = _deg_kernel(dst2)
    hs0 = _tc1(x, W0, degp)
    acc0 = _agg_kernel(src2, dst2, hs0)
    hs1 = _tc2(acc0, hs0, degp, b0, W1)
    acc1 = _agg_kernel(src2, dst2, hs1)
    return _tc3(acc1, hs1, degp, b1)


# speedup vs baseline: 4.3277x; 1.0807x over previous
import functools

import jax
import jax.numpy as jnp
from jax import lax
from jax.experimental import pallas as pl
from jax.experimental.pallas import tpu as pltpu
from jax.experimental.pallas import tpu_sc as plsc

N = 10000
E = 320000
D = 128
NC = 2
NS = 16
LANES = 16
CH = 128
E_PAD = 327680
ROWS_TOT = E_PAD // CH
NR = 16
N_PAD = 10240
PAD_DST = N
A_RPT = N_PAD // NS
ROWS_PT = ROWS_TOT // (NC * NS)
BLKS = ROWS_PT // NR
C0_ROWS_PT = 80
C1_ROWS_PT = 80
C0_ROWS = C0_ROWS_PT * NS

_mesh = plsc.VectorSubcoreMesh(
    core_axis_name="c", subcore_axis_name="s", num_cores=NC, num_subcores=NS)


@functools.partial(
    pl.kernel,
    out_type=jax.ShapeDtypeStruct((NC, N_PAD, LANES), jnp.float32),
    mesh=_mesh,
    scratch_types=[
        pltpu.VMEM_SHARED((N_PAD, LANES), jnp.float32),
        pltpu.VMEM((NR, CH), jnp.int32),
        pltpu.VMEM((CH, LANES), jnp.float32),
        pltpu.SemaphoreType.DMA,
    ],
)
def _deg_kernel(dst_hbm, out_hbm, hist, didx, ones, ssem):
    c = lax.axis_index("c")
    s = lax.axis_index("s")
    t = c * NS + s
    one16 = jnp.full((LANES,), 1.0, jnp.float32)
    zero16 = jnp.zeros((LANES,), jnp.float32)

    def zfill(i, _):
        ones[i, :] = zero16
        return 0

    lax.fori_loop(0, CH, zfill, 0)
    for k in range(A_RPT // CH):
        pltpu.sync_copy(ones, hist.at[pl.ds(s * A_RPT + k * CH, CH)])

    def ofill(i, _):
        ones[i, :] = one16
        return 0

    lax.fori_loop(0, CH, ofill, 0)
    plsc.subcore_barrier()

    def blk(j, _):
        row0 = t * ROWS_PT + j * NR
        pltpu.sync_copy(dst_hbm.at[pl.ds(row0, NR)], didx)
        cps = [pltpu.async_copy(ones, hist.at[didx.at[r]], ssem, add=True)
               for r in range(NR)]
        for cp in cps:
            cp.wait()
        return 0

    lax.fori_loop(0, BLKS, blk, 0)
    plsc.subcore_barrier()

    off = s * A_RPT
    pltpu.sync_copy(hist.at[pl.ds(off, A_RPT)],
                    out_hbm.at[c, pl.ds(off, A_RPT)])


@functools.partial(
    pl.kernel,
    out_type=jax.ShapeDtypeStruct((NC, N_PAD, D), jnp.float32),
    mesh=_mesh,
    scratch_types=[
        pltpu.VMEM_SHARED((N_PAD, D), jnp.float32),
        pltpu.VMEM((NR, CH), jnp.int32),
        pltpu.VMEM((NR, CH), jnp.int32),
        pltpu.VMEM((CH, D), jnp.float32),
        pltpu.VMEM((CH, D), jnp.float32),
        pltpu.SemaphoreType.DMA,
        pltpu.SemaphoreType.DMA,
        pltpu.SemaphoreType.DMA,
        pltpu.SemaphoreType.DMA,
    ],
)
def _agg_kernel(src_hbm, dst_hbm, hs_hbm, out_hbm, acc, sidx, didx, rows0,
                rows1, gsem0, gsem1, ssem0, ssem1):
    c = lax.axis_index("c")
    s = lax.axis_index("s")
    zero16 = jnp.zeros((LANES,), jnp.float32)
    rows = (rows0, rows1)
    gsems = (gsem0, gsem1)
    ssems = (ssem0, ssem1)
    NB = 2

    def zfill(i, _):
        for k in range(D // LANES):
            rows0[i, k * LANES:(k + 1) * LANES] = zero16
        return 0

    lax.fori_loop(0, CH, zfill, 0)
    for k in range(A_RPT // CH):
        pltpu.sync_copy(rows0, acc.at[pl.ds(s * A_RPT + k * CH, CH)])
    plsc.subcore_barrier()

    tile_base = jnp.where(c == 0, s * C0_ROWS_PT, C0_ROWS + s * C1_ROWS_PT)
    nblk = jnp.where(c == 0, C0_ROWS_PT // NR, C1_ROWS_PT // NR)

    def blk(j, _):
        row0 = tile_base + j * NR
        pltpu.sync_copy(src_hbm.at[pl.ds(row0, NR)], sidx)
        pltpu.sync_copy(dst_hbm.at[pl.ds(row0, NR)], didx)
        gd = {}
        sd = {}
        gd[0] = pltpu.async_copy(hs_hbm.at[sidx.at[0]], rows[0], gsems[0])
        for r in range(NR):
            if r + 1 < NR:
                nb = (r + 1) % NB
                if r >= 1:
                    sd[r - 1].wait()
                gd[r + 1] = pltpu.async_copy(hs_hbm.at[sidx.at[r + 1]],
                                             rows[nb], gsems[nb])
            gd[r].wait()
            b = r % NB
            sd[r] = pltpu.async_copy(rows[b], acc.at[didx.at[r]], ssems[b],
                                     add=True)
        for r in range(max(0, NR - NB), NR):
            sd[r].wait()
        return 0

    lax.fori_loop(0, nblk, blk, 0)
    plsc.subcore_barrier()

    o = s * A_RPT
    pltpu.sync_copy(acc.at[pl.ds(o, A_RPT)], out_hbm.at[c, pl.ds(o, A_RPT)])


_BN = 2000


def _dinv_of(degp_ref):
    deg = degp_ref[0, :, 0] + degp_ref[1, :, 0] + 1.0
    return lax.rsqrt(deg)


def _tc1_body(x_ref, w_ref, degp_ref, out_ref):
    dinv = _dinv_of(degp_ref)
    h = jnp.dot(x_ref[...], w_ref[...], preferred_element_type=jnp.float32)
    out_ref[...] = h * dinv[:, None]


def _tc2_body(acc_ref, hs_ref, degp_ref, b_ref, w_ref, out_ref):
    dinv = _dinv_of(degp_ref)
    tot = acc_ref[0] + acc_ref[1] + hs_ref[...]
    z = jnp.maximum(dinv[:, None] * tot + b_ref[...][None, :], 0.0)
    h = jnp.dot(z, w_ref[...], preferred_element_type=jnp.float32)
    out_ref[...] = h * dinv[:, None]


def _tc3_body(acc_ref, hs_ref, degp_ref, b_ref, out_ref):
    dinv = _dinv_of(degp_ref)
    tot = acc_ref[0] + acc_ref[1] + hs_ref[...]
    out_ref[...] = jnp.maximum(dinv[:, None] * tot + b_ref[...][None, :], 0.0)


_acc_spec = pl.BlockSpec((NC, _BN, D), lambda i: (0, i, 0))
_hs_spec = pl.BlockSpec((_BN, D), lambda i: (i, 0))
_degp_spec = pl.BlockSpec((NC, _BN, LANES), lambda i: (0, i, 0))
_vec_spec = pl.BlockSpec((D,), lambda i: (0,))
_w_spec = pl.BlockSpec((D, D), lambda i: (0, 0))


def _tc1(x, w0, degp):
    return pl.pallas_call(
        _tc1_body,
        grid=(N // _BN,),
        in_specs=[_hs_spec, _w_spec, _degp_spec],
        out_specs=_hs_spec,
        out_shape=jax.ShapeDtypeStruct((N, D), jnp.float32),
    )(x, w0, degp)


def _tc2(acc0, hs0, degp, b0, w1):
    return pl.pallas_call(
        _tc2_body,
        grid=(N // _BN,),
        in_specs=[_acc_spec, _hs_spec, _degp_spec, _vec_spec, _w_spec],
        out_specs=_hs_spec,
        out_shape=jax.ShapeDtypeStruct((N, D), jnp.float32),
    )(acc0, hs0, degp, b0, w1)


def _tc3(acc1, hs1, degp, b1):
    return pl.pallas_call(
        _tc3_body,
        grid=(N // _BN,),
        in_specs=[_acc_spec, _hs_spec, _degp_spec, _vec_spec],
        out_specs=_hs_spec,
        out_shape=jax.ShapeDtypeStruct((N, D), jnp.float32),
    )(acc1, hs1, degp, b1)


def kernel(x, edge_index, W0, b0, W1, b1):
    ei = edge_index.astype(jnp.int32)
    pad = E_PAD - E
    src2 = jnp.concatenate([ei[0], jnp.arange(pad, dtype=jnp.int32)]).reshape(
        ROWS_TOT, CH)
    padv = N + jnp.arange(pad, dtype=jnp.int32) % (N_PAD - N)
    dst2 = jnp.concatenate([ei[1], padv]).reshape(ROWS_TOT, CH)
    degp = _deg_kernel(dst2)
    hs0 = _tc1(x, W0, degp)
    acc0 = _agg_kernel(src2, dst2, hs0)
    hs1 = _tc2(acc0, hs0, degp, b0, W1)
    acc1 = _agg_kernel(src2, dst2, hs1)
    return _tc3(acc1, hs1, degp, b1)
